# baseline (speedup 1.0000x reference)
import functools

import jax
import jax.numpy as jnp
from jax import lax
from jax.experimental import pallas as pl
from jax.experimental.pallas import tpu as pltpu
from jax.experimental.pallas import tpu_sc as plsc

POOL = 16
B, C, L = 4, 768, 4096
NUM_WORKERS = 32
W_PER_B = NUM_WORKERS // B
ROWS_PER_W = C // W_PER_B
CHUNK_ROWS = 8
NCHUNKS = ROWS_PER_W // CHUNK_ROWS
NSLOTS = 3


def _extrema_body(x_hbm, out_hbm, b0, b1, b2, si0, si1, si2, so0, so1, so2):
    cid = lax.axis_index("c")
    sid = lax.axis_index("s")
    wid = sid * 2 + cid
    b_idx = wid // W_PER_B
    row_base = (wid % W_PER_B) * ROWS_PER_W
    lanes = lax.iota(jnp.int32, POOL)
    bufs = (b0, b1, b2)
    sis, sos = (si0, si1, si2), (so0, so1, so2)

    def in_copy(ci, s):
        return pltpu.make_async_copy(
            x_hbm.at[b_idx, pl.ds(row_base + ci * CHUNK_ROWS, CHUNK_ROWS), :],
            bufs[s], sis[s])

    def out_copy(ci, s):
        return pltpu.make_async_copy(
            bufs[s],
            out_hbm.at[b_idx, pl.ds(row_base + ci * CHUNK_ROWS, CHUNK_ROWS), :],
            sos[s])

    in_copy(0, 0).start()
    in_copy(1, 1).start()

    def tri_body(p, carry):
        for s in range(NSLOTS):
            ci = NSLOTS * p + s
            in_copy(ci, s).wait()

            for r in range(CHUNK_ROWS):
                @plsc.parallel_loop(0, L, step=POOL, unroll=16)
                def win_body(coff):
                    w = bufs[s][r, pl.ds(coff, POOL)]
                    a = jnp.abs(w)
                    mx = jnp.max(a)
                    first = plsc.all_reduce_ffs(a == mx)
                    bufs[s][r, pl.ds(coff, POOL)] = jnp.where(
                        lanes == first, w, 0.0)

            out_copy(ci, s).start()

            s2 = (s + 2) % NSLOTS
            @pl.when(ci + 2 < NCHUNKS)
            def _():
                @pl.when(ci >= 1)
                def _():
                    out_copy(ci - 1, s2).wait()

                in_copy(ci + 2, s2).start()

        return carry

    lax.fori_loop(0, NCHUNKS // NSLOTS, tri_body, 0)
    out_copy(NCHUNKS - 3, (NCHUNKS - 3) % NSLOTS).wait()
    out_copy(NCHUNKS - 2, (NCHUNKS - 2) % NSLOTS).wait()
    out_copy(NCHUNKS - 1, (NCHUNKS - 1) % NSLOTS).wait()


def kernel(input_):
    mesh = plsc.VectorSubcoreMesh(core_axis_name="c", subcore_axis_name="s")
    return pl.kernel(
        _extrema_body,
        mesh=mesh,
        out_type=jax.ShapeDtypeStruct((B, C, L), jnp.float32),
        scratch_types=[
            pltpu.VMEM((CHUNK_ROWS, L), jnp.float32),
            pltpu.VMEM((CHUNK_ROWS, L), jnp.float32),
            pltpu.VMEM((CHUNK_ROWS, L), jnp.float32),
            pltpu.SemaphoreType.DMA,
            pltpu.SemaphoreType.DMA,
            pltpu.SemaphoreType.DMA,
            pltpu.SemaphoreType.DMA,
            pltpu.SemaphoreType.DMA,
            pltpu.SemaphoreType.DMA,
        ],
        compiler_params=pltpu.CompilerParams(needs_layout_passes=False),
    )(input_)
